# baseline (speedup 1.0000x reference)
import functools

import jax
import jax.numpy as jnp
from jax import lax
from jax.experimental import pallas as pl
from jax.experimental.pallas import tpu as pltpu
from jax.experimental.pallas import tpu_sc as plsc

_N_PATCH = 4096
_N_GT = 524288
_N_Q = 16384
_N_TILES = 16
_MAP_SIZE = _N_PATCH * _N_PATCH
_PAD_HALF = 65536
_TABLE_SIZE = _MAP_SIZE + 2 * _PAD_HALF

_GT_PER_TILE = _N_GT // _N_TILES
_Q_PER_TILE = _N_Q // _N_TILES
_GT_CHUNK = 8192
_N_GT_CHUNKS = _GT_PER_TILE // _GT_CHUNK

_ACCEPTANCE_OVERLAP = 0.3
_RRE_THRESHOLD = 5.0
_RTE_THRESHOLD = 2.0


def _sc_body(gt_r, gt_s, gt_ov, q_r, q_s, partials,
             qrb, qsb, qkb, qvb, rb, sb, ovb, kba, kbb, ones_v, zeros_v,
             prow, table, sem):
    sid = lax.axis_index("s")
    cid = lax.axis_index("c")

    for l in range(8):
        zeros_v[pl.ds(l * 16, 16)] = jnp.zeros((16,), jnp.float32)
        ones_v[pl.ds(l * 16, 16)] = jnp.ones((16,), jnp.float32)

    pad_base = _MAP_SIZE + cid * _PAD_HALF
    qbase = sid * _Q_PER_TILE
    pltpu.sync_copy(q_r.at[pl.ds(qbase, _Q_PER_TILE)], qrb)
    pltpu.sync_copy(q_s.at[pl.ds(qbase, _Q_PER_TILE)], qsb)
    for j in range(8):
        for l in range(8):
            o = j * 128 + l * 16
            qk = qrb[pl.ds(o, 16)] * _N_PATCH + qsb[pl.ds(o, 16)]
            qk = jnp.where((qk >> 23) == cid, qk,
                           pad_base + (qk & (_PAD_HALF - 1)))
            qkb[j, pl.ds(l * 16, 16)] = qk
    for j in range(8):
        pltpu.async_copy(zeros_v, table.at[qkb.at[j]], sem)
    for j in range(8):
        pltpu.make_async_copy(zeros_v, table.at[qkb.at[j]], sem).wait()

    plsc.subcore_barrier()

    n_rows = _GT_CHUNK // 128

    def fire(kb):
        def body(j, _):
            pltpu.async_copy(
                ones_v, table.at[plsc.Indices(kb.at[j], ignored_value=-1)],
                sem)
            return ()
        lax.fori_loop(0, n_rows, body, (), unroll=False)

    def drain(kb):
        def body(j, _):
            pltpu.make_async_copy(
                ones_v, table.at[plsc.Indices(kb.at[j], ignored_value=-1)],
                sem).wait()
            return ()
        lax.fori_loop(0, n_rows, body, (), unroll=False)

    prev_kb = None
    for c in range(_N_GT_CHUNKS):
        kb = kba if c % 2 == 0 else kbb
        gbase = sid * _GT_PER_TILE + c * _GT_CHUNK
        pltpu.sync_copy(gt_r.at[pl.ds(gbase, _GT_CHUNK)], rb)
        pltpu.sync_copy(gt_s.at[pl.ds(gbase, _GT_CHUNK)], sb)
        pltpu.sync_copy(gt_ov.at[pl.ds(gbase, _GT_CHUNK)], ovb)

        def compute_body(j, _):
            for l in range(8):
                o = j * 128 + l * 16
                rv = rb[pl.ds(o, 16)]
                sv = sb[pl.ds(o, 16)]
                ov = ovb[pl.ds(o, 16)]
                key = rv * _N_PATCH + sv
                keep = jnp.logical_and(ov > _ACCEPTANCE_OVERLAP,
                                       (key >> 23) == cid)
                kb[j, pl.ds(l * 16, 16)] = jnp.where(keep, key, -1)
            return ()

        lax.fori_loop(0, n_rows, compute_body, (), unroll=False)
        if prev_kb is not None:
            drain(prev_kb)
        fire(kb)
        prev_kb = kb
    drain(prev_kb)

    plsc.subcore_barrier()

    for j in range(8):
        pltpu.async_copy(table.at[qkb.at[j]], qvb.at[j], sem)
    for j in range(8):
        pltpu.make_async_copy(table.at[qkb.at[j]], qvb.at[j], sem).wait()

    acc = jnp.zeros((16,), jnp.float32)
    for j in range(8):
        for l in range(8):
            acc = acc + qvb[j, pl.ds(l * 16, 16)]
    prow[...] = acc
    pltpu.sync_copy(prow, partials.at[sid * 2 + cid])


@jax.jit
def _sc_scatter_gather(gt_r, gt_s, gt_ov, q_r, q_s):
    mesh = plsc.VectorSubcoreMesh(
        core_axis_name="c", subcore_axis_name="s")
    f = pl.kernel(
        _sc_body,
        out_type=jax.ShapeDtypeStruct((2 * _N_TILES, 16), jnp.float32),
        mesh=mesh,
        scratch_types=[
            pltpu.VMEM((_Q_PER_TILE,), jnp.int32),
            pltpu.VMEM((_Q_PER_TILE,), jnp.int32),
            pltpu.VMEM((8, 128), jnp.int32),
            pltpu.VMEM((8, 128), jnp.float32),
            pltpu.VMEM((_GT_CHUNK,), jnp.int32),
            pltpu.VMEM((_GT_CHUNK,), jnp.int32),
            pltpu.VMEM((_GT_CHUNK,), jnp.float32),
            pltpu.VMEM((_GT_CHUNK // 128, 128), jnp.int32),
            pltpu.VMEM((_GT_CHUNK // 128, 128), jnp.int32),
            pltpu.VMEM((128,), jnp.float32),
            pltpu.VMEM((128,), jnp.float32),
            pltpu.VMEM((16,), jnp.float32),
            pltpu.HBM((_TABLE_SIZE,), jnp.float32),
            pltpu.SemaphoreType.DMA,
        ],
    )
    return f(gt_r, gt_s, gt_ov, q_r, q_s)


def _tc_metrics_body(part_ref, gt_ref, tf_ref, rtf_ref, out_ref):
    pir = jnp.sum(part_ref[...]) * (1.0 / _N_Q)
    pmr = (pir > 0.2).astype(jnp.float32)

    g33 = gt_ref[0:3, 0:3]
    gt3 = gt_ref[0:3, 3]

    def terr(t_ref):
        t33 = t_ref[0:3, 0:3]
        cosv = jnp.clip(0.5 * (jnp.sum(t33 * g33) - 1.0), -1.0, 1.0)
        acosv = 2.0 * jnp.arctan2(jnp.sqrt(1.0 - cosv), jnp.sqrt(1.0 + cosv))
        rre = 180.0 / jnp.pi * acosv
        d = gt3 - t_ref[0:3, 3]
        rte = jnp.sqrt(jnp.sum(d * d))
        return rte, rre

    te, re = terr(tf_ref)
    rte, rre = terr(rtf_ref)
    rr = jnp.logical_and(rre < _RRE_THRESHOLD,
                         rte < _RTE_THRESHOLD).astype(jnp.float32)
    lane = lax.broadcasted_iota(jnp.int32, (8,), 0)
    out = jnp.where(lane == 0, pir, 0.0)
    out = jnp.where(lane == 1, pmr, out)
    out = jnp.where(lane == 2, re, out)
    out = jnp.where(lane == 3, te, out)
    out = jnp.where(lane == 4, rre, out)
    out = jnp.where(lane == 5, rte, out)
    out = jnp.where(lane == 6, rr, out)
    out_ref[...] = out


@jax.jit
def _tc_metrics(partials, gt_transform, transform, refined_transform):
    return pl.pallas_call(
        _tc_metrics_body,
        out_shape=jax.ShapeDtypeStruct((8,), jnp.float32),
    )(partials, gt_transform, transform, refined_transform)


def kernel(ref_feats_c, src_feats_c, gt_patch_corr_overlaps,
           gt_patch_corr_indices, ref_patch_corr_indices,
           src_patch_corr_indices, gt_transform, transform,
           refined_transform):
    gt_r = gt_patch_corr_indices[:, 0].astype(jnp.int32)
    gt_s = gt_patch_corr_indices[:, 1].astype(jnp.int32)
    q_r = ref_patch_corr_indices.astype(jnp.int32)
    q_s = src_patch_corr_indices.astype(jnp.int32)
    partials = _sc_scatter_gather(gt_r, gt_s, gt_patch_corr_overlaps,
                                  q_r, q_s)
    out = _tc_metrics(partials, gt_transform, transform, refined_transform)
    return out[:7]


# speedup vs baseline: 1.0516x; 1.0516x over previous
import functools

import jax
import jax.numpy as jnp
from jax import lax
from jax.experimental import pallas as pl
from jax.experimental.pallas import tpu as pltpu
from jax.experimental.pallas import tpu_sc as plsc

_N_PATCH = 4096
_N_GT = 524288
_N_Q = 16384
_N_TILES = 16
_MAP_SIZE = _N_PATCH * _N_PATCH
_PAD_HALF = 65536
_TABLE_SIZE = _MAP_SIZE + 2 * _PAD_HALF

_GT_PER_TILE = _N_GT // _N_TILES
_Q_PER_TILE = _N_Q // _N_TILES
_GT_CHUNK = 8192
_N_GT_CHUNKS = _GT_PER_TILE // _GT_CHUNK

_ACCEPTANCE_OVERLAP = 0.3
_RRE_THRESHOLD = 5.0
_RTE_THRESHOLD = 2.0


def _sc_body(gt_r, gt_s, gt_ov, q_r, q_s, partials,
             qrb, qsb, qkb, qvb, rb, sb, ovb, kba, kbb, ones_v, zeros_v,
             prow, table, sem):
    sid = lax.axis_index("s")

    for l in range(8):
        zeros_v[pl.ds(l * 16, 16)] = jnp.zeros((16,), jnp.float32)
        ones_v[pl.ds(l * 16, 16)] = jnp.ones((16,), jnp.float32)

    qbase = sid * _Q_PER_TILE
    pltpu.sync_copy(q_r.at[pl.ds(qbase, _Q_PER_TILE)], qrb)
    pltpu.sync_copy(q_s.at[pl.ds(qbase, _Q_PER_TILE)], qsb)
    for j in range(8):
        for l in range(8):
            o = j * 128 + l * 16
            qk = qrb[pl.ds(o, 16)] * _N_PATCH + qsb[pl.ds(o, 16)]
            qkb[j, pl.ds(l * 16, 16)] = qk
    for j in range(8):
        pltpu.async_copy(zeros_v, table.at[qkb.at[j]], sem)
    for j in range(8):
        pltpu.make_async_copy(zeros_v, table.at[qkb.at[j]], sem).wait()

    plsc.subcore_barrier()

    n_rows = _GT_CHUNK // 128

    def fire(kb):
        def body(j, _):
            pltpu.async_copy(
                ones_v, table.at[plsc.Indices(kb.at[j], ignored_value=-1)],
                sem)
            return ()
        lax.fori_loop(0, n_rows, body, (), unroll=False)

    def drain(kb):
        def body(j, _):
            pltpu.make_async_copy(
                ones_v, table.at[plsc.Indices(kb.at[j], ignored_value=-1)],
                sem).wait()
            return ()
        lax.fori_loop(0, n_rows, body, (), unroll=False)

    prev_kb = None
    for c in range(_N_GT_CHUNKS):
        kb = kba if c % 2 == 0 else kbb
        gbase = sid * _GT_PER_TILE + c * _GT_CHUNK
        pltpu.sync_copy(gt_r.at[pl.ds(gbase, _GT_CHUNK)], rb)
        pltpu.sync_copy(gt_s.at[pl.ds(gbase, _GT_CHUNK)], sb)
        pltpu.sync_copy(gt_ov.at[pl.ds(gbase, _GT_CHUNK)], ovb)

        def compute_body(j, _):
            for l in range(8):
                o = j * 128 + l * 16
                rv = rb[pl.ds(o, 16)]
                sv = sb[pl.ds(o, 16)]
                ov = ovb[pl.ds(o, 16)]
                key = rv * _N_PATCH + sv
                kb[j, pl.ds(l * 16, 16)] = jnp.where(
                    ov > _ACCEPTANCE_OVERLAP, key, -1)
            return ()

        lax.fori_loop(0, n_rows, compute_body, (), unroll=False)
        if prev_kb is not None:
            drain(prev_kb)
        fire(kb)
        prev_kb = kb
    drain(prev_kb)

    plsc.subcore_barrier()

    for j in range(8):
        pltpu.async_copy(table.at[qkb.at[j]], qvb.at[j], sem)
    for j in range(8):
        pltpu.make_async_copy(table.at[qkb.at[j]], qvb.at[j], sem).wait()

    acc = jnp.zeros((16,), jnp.float32)
    for j in range(8):
        for l in range(8):
            acc = acc + qvb[j, pl.ds(l * 16, 16)]
    prow[...] = acc
    pltpu.sync_copy(prow, partials.at[sid])


@jax.jit
def _sc_scatter_gather(gt_r, gt_s, gt_ov, q_r, q_s):
    mesh = plsc.VectorSubcoreMesh(
        core_axis_name="c", subcore_axis_name="s", num_cores=1)
    f = pl.kernel(
        _sc_body,
        out_type=jax.ShapeDtypeStruct((_N_TILES, 16), jnp.float32),
        mesh=mesh,
        scratch_types=[
            pltpu.VMEM((_Q_PER_TILE,), jnp.int32),
            pltpu.VMEM((_Q_PER_TILE,), jnp.int32),
            pltpu.VMEM((8, 128), jnp.int32),
            pltpu.VMEM((8, 128), jnp.float32),
            pltpu.VMEM((_GT_CHUNK,), jnp.int32),
            pltpu.VMEM((_GT_CHUNK,), jnp.int32),
            pltpu.VMEM((_GT_CHUNK,), jnp.float32),
            pltpu.VMEM((_GT_CHUNK // 128, 128), jnp.int32),
            pltpu.VMEM((_GT_CHUNK // 128, 128), jnp.int32),
            pltpu.VMEM((128,), jnp.float32),
            pltpu.VMEM((128,), jnp.float32),
            pltpu.VMEM((16,), jnp.float32),
            pltpu.HBM((_TABLE_SIZE,), jnp.float32),
            pltpu.SemaphoreType.DMA,
        ],
    )
    return f(gt_r, gt_s, gt_ov, q_r, q_s)


def _tc_metrics_body(part_ref, gt_ref, tf_ref, rtf_ref, out_ref):
    pir = jnp.sum(part_ref[...]) * (1.0 / _N_Q)
    pmr = (pir > 0.2).astype(jnp.float32)

    g33 = gt_ref[0:3, 0:3]
    gt3 = gt_ref[0:3, 3]

    def terr(t_ref):
        t33 = t_ref[0:3, 0:3]
        cosv = jnp.clip(0.5 * (jnp.sum(t33 * g33) - 1.0), -1.0, 1.0)
        acosv = 2.0 * jnp.arctan2(jnp.sqrt(1.0 - cosv), jnp.sqrt(1.0 + cosv))
        rre = 180.0 / jnp.pi * acosv
        d = gt3 - t_ref[0:3, 3]
        rte = jnp.sqrt(jnp.sum(d * d))
        return rte, rre

    te, re = terr(tf_ref)
    rte, rre = terr(rtf_ref)
    rr = jnp.logical_and(rre < _RRE_THRESHOLD,
                         rte < _RTE_THRESHOLD).astype(jnp.float32)
    lane = lax.broadcasted_iota(jnp.int32, (8,), 0)
    out = jnp.where(lane == 0, pir, 0.0)
    out = jnp.where(lane == 1, pmr, out)
    out = jnp.where(lane == 2, re, out)
    out = jnp.where(lane == 3, te, out)
    out = jnp.where(lane == 4, rre, out)
    out = jnp.where(lane == 5, rte, out)
    out = jnp.where(lane == 6, rr, out)
    out_ref[...] = out


@jax.jit
def _tc_metrics(partials, gt_transform, transform, refined_transform):
    return pl.pallas_call(
        _tc_metrics_body,
        out_shape=jax.ShapeDtypeStruct((8,), jnp.float32),
    )(partials, gt_transform, transform, refined_transform)


def kernel(ref_feats_c, src_feats_c, gt_patch_corr_overlaps,
           gt_patch_corr_indices, ref_patch_corr_indices,
           src_patch_corr_indices, gt_transform, transform,
           refined_transform):
    gt_r = gt_patch_corr_indices[:, 0].astype(jnp.int32)
    gt_s = gt_patch_corr_indices[:, 1].astype(jnp.int32)
    q_r = ref_patch_corr_indices.astype(jnp.int32)
    q_s = src_patch_corr_indices.astype(jnp.int32)
    partials = _sc_scatter_gather(gt_r, gt_s, gt_patch_corr_overlaps,
                                  q_r, q_s)
    out = _tc_metrics(partials, gt_transform, transform, refined_transform)
    return out[:7]
